# baseline (speedup 1.0000x reference)
import jax
import jax.numpy as jnp
from jax import lax
from jax.experimental import pallas as pl
from jax.experimental.pallas import tpu as pltpu
from jax.experimental.pallas import tpu_sc as plsc

_D = 1024
_B, _T = 4, 2048
_N = _B * _T
_NC, _NS = 2, 16
_NW = _NC * _NS
_TPW = _T // _NW
_C = 32
_NH = _TPW // _C
_NCHUNK = _NH * _B
_SL = _D // 16


def _body(idx_hbm, emb_hbm, pos_hbm, out_hbm,
          idx_v, pbuf, g0, g1, sg0, sg1, so0, so1):
    wid = lax.axis_index("s") * _NC + lax.axis_index("c")
    t0 = wid * _TPW

    pltpu.sync_copy(idx_hbm.at[wid], idx_v)

    gs = (g0, g1)
    sgs = (sg0, sg1)
    sos = (so0, so1)
    cp_out = [None] * _NCHUNK

    def start(k):
        slot = k % 2
        if k >= 2:
            cp_out[k - 2].wait()
        return pltpu.async_copy(emb_hbm.at[idx_v.at[k]], gs[slot], sgs[slot])

    def process(k, cp_g):
        slot = k % 2
        g = gs[slot]
        cp_g.wait()

        def add_row(r, _):
            for c in range(_SL):
                sl = pl.ds(c * 16, 16)
                plsc.addupdate(g.at[r, sl], pbuf[r, sl])
            return 0

        lax.fori_loop(0, _C, add_row, 0)
        h, b = divmod(k, _B)
        dst = out_hbm.at[pl.ds(b * _T + t0 + h * _C, _C)]
        cp_out[k] = pltpu.async_copy(g, dst, sos[slot])

    pltpu.sync_copy(pos_hbm.at[pl.ds(t0, _C)], pbuf)
    cp = start(0)
    for k in range(1, _NCHUNK):
        cp_next = start(k)
        process(k - 1, cp)
        if k == _B:
            pltpu.sync_copy(pos_hbm.at[pl.ds(t0 + _C, _C)], pbuf)
        cp = cp_next
    process(_NCHUNK - 1, cp)
    cp_out[_NCHUNK - 2].wait()
    cp_out[_NCHUNK - 1].wait()


@jax.jit
def _run(idx, embedding, position_embedding):
    mesh = plsc.VectorSubcoreMesh(
        core_axis_name="c", subcore_axis_name="s", num_cores=_NC,
        num_subcores=_NS)
    out = pl.kernel(
        _body,
        out_type=jax.ShapeDtypeStruct((_N, _D), jnp.float32),
        mesh=mesh,
        scratch_types=[
            pltpu.VMEM((_NCHUNK, _C), jnp.int32),
            pltpu.VMEM((_C, _D), jnp.float32),
            pltpu.VMEM((_C, _D), jnp.float32),
            pltpu.VMEM((_C, _D), jnp.float32),
            pltpu.SemaphoreType.DMA,
            pltpu.SemaphoreType.DMA,
            pltpu.SemaphoreType.DMA,
            pltpu.SemaphoreType.DMA,
        ],
        name="emb_lookup_sc",
    )(idx, embedding, position_embedding)
    return out.reshape(_B, _T, _D)


def kernel(x, embedding, position_embedding):
    idx = (x.astype(jnp.int32)
           .reshape(_B, _NW, _NH, _C)
           .transpose(1, 2, 0, 3)
           .reshape(_NW, _NCHUNK, _C))
    return _run(idx, embedding, position_embedding)


# speedup vs baseline: 1.6488x; 1.6488x over previous
import jax
import jax.numpy as jnp
from jax import lax
from jax.experimental import pallas as pl
from jax.experimental.pallas import tpu as pltpu
from jax.experimental.pallas import tpu_sc as plsc

_D = 1024
_B, _T = 4, 2048
_N = _B * _T
_NC, _NS = 2, 16
_NW = _NC * _NS
_TPW = _T // _NW
_C = 32
_NH = _TPW // _C
_NCHUNK = _NH * _B
_SL = _D // 16


def _body(idx_hbm, emb_hbm, pos_hbm, out_hbm,
          idx_v, pbuf, g0, g1, sg0, sg1, so0, so1):
    wid = lax.axis_index("s") * _NC + lax.axis_index("c")
    t0 = wid * _TPW

    pltpu.sync_copy(idx_hbm.at[wid], idx_v)

    gs = (g0, g1)
    sgs = (sg0, sg1)
    sos = (so0, so1)
    cp_out = [None] * _NCHUNK

    def start(k):
        slot = k % 2
        if k >= 2:
            cp_out[k - 2].wait()
        return pltpu.async_copy(emb_hbm.at[idx_v.at[k]], gs[slot], sgs[slot])

    def process(k, cp_g):
        slot = k % 2
        g = gs[slot]
        cp_g.wait()

        @plsc.parallel_loop(0, _C, 1, unroll=1)
        def add_row(r):
            for c in range(_SL):
                sl = pl.ds(c * 16, 16)
                g[r, sl] = g[r, sl] + pbuf[r, sl]
        h, b = divmod(k, _B)
        dst = out_hbm.at[pl.ds(b * _T + t0 + h * _C, _C)]
        cp_out[k] = pltpu.async_copy(g, dst, sos[slot])

    pltpu.sync_copy(pos_hbm.at[pl.ds(t0, _C)], pbuf)
    cp = start(0)
    for k in range(1, _NCHUNK):
        cp_next = start(k)
        process(k - 1, cp)
        if k == _B:
            pltpu.sync_copy(pos_hbm.at[pl.ds(t0 + _C, _C)], pbuf)
        cp = cp_next
    process(_NCHUNK - 1, cp)
    cp_out[_NCHUNK - 2].wait()
    cp_out[_NCHUNK - 1].wait()


@jax.jit
def _run(idx, embedding, position_embedding):
    mesh = plsc.VectorSubcoreMesh(
        core_axis_name="c", subcore_axis_name="s", num_cores=_NC,
        num_subcores=_NS)
    out = pl.kernel(
        _body,
        out_type=jax.ShapeDtypeStruct((_N, _D), jnp.float32),
        mesh=mesh,
        scratch_types=[
            pltpu.VMEM((_NCHUNK, _C), jnp.int32),
            pltpu.VMEM((_C, _D), jnp.float32),
            pltpu.VMEM((_C, _D), jnp.float32),
            pltpu.VMEM((_C, _D), jnp.float32),
            pltpu.SemaphoreType.DMA,
            pltpu.SemaphoreType.DMA,
            pltpu.SemaphoreType.DMA,
            pltpu.SemaphoreType.DMA,
        ],
        name="emb_lookup_sc",
    )(idx, embedding, position_embedding)
    return out.reshape(_B, _T, _D)


def kernel(x, embedding, position_embedding):
    idx = (x.astype(jnp.int32)
           .reshape(_B, _NW, _NH, _C)
           .transpose(1, 2, 0, 3)
           .reshape(_NW, _NCHUNK, _C))
    return _run(idx, embedding, position_embedding)


# speedup vs baseline: 1.6868x; 1.0231x over previous
import jax
import jax.numpy as jnp
from jax import lax
from jax.experimental import pallas as pl
from jax.experimental.pallas import tpu as pltpu
from jax.experimental.pallas import tpu_sc as plsc

_D = 1024
_B, _T = 4, 2048
_N = _B * _T
_NC, _NS = 2, 16
_NW = _NC * _NS
_TPW = _T // _NW
_C = 16
_NQ = _TPW // _C
_NCHUNK = _NQ * _B
_SL = _D // 16
_NG = 4


def _body(idx_hbm, emb_hbm, pos_hbm, out_hbm,
          idx_v, p0, p1, g0, g1, g2, g3,
          sp0, sp1, sg0, sg1, sg2, sg3, so0, so1, so2, so3):
    wid = lax.axis_index("s") * _NC + lax.axis_index("c")
    t0 = wid * _TPW

    pltpu.sync_copy(idx_hbm.at[wid], idx_v)

    ps = (p0, p1)
    sps = (sp0, sp1)
    gs = (g0, g1, g2, g3)
    sgs = (sg0, sg1, sg2, sg3)
    sos = (so0, so1, so2, so3)
    cp_out = [None] * _NCHUNK
    cp_pos = [None] * _NQ

    pltpu.sync_copy(pos_hbm.at[pl.ds(t0, _C)], p0)
    cp_pos[1] = pltpu.async_copy(pos_hbm.at[pl.ds(t0 + _C, _C)], p1, sp1)

    def start(k):
        slot = k % _NG
        if k >= _NG:
            cp_out[k - _NG].wait()
        return pltpu.async_copy(emb_hbm.at[idx_v.at[k]], gs[slot], sgs[slot])

    def process(k, cp_g):
        q, b = divmod(k, _B)
        slot = k % _NG
        g = gs[slot]
        if b == 0 and q > 0:
            cp_pos[q].wait()
        cp_g.wait()
        pb = ps[q % 2]

        @plsc.parallel_loop(0, _C, 1, unroll=1)
        def add_row(r):
            for c in range(_SL):
                sl = pl.ds(c * 16, 16)
                g[r, sl] = g[r, sl] + pb[r, sl]

        dst = out_hbm.at[pl.ds(b * _T + t0 + q * _C, _C)]
        cp_out[k] = pltpu.async_copy(g, dst, sos[slot])
        if b == _B - 1 and q + 2 < _NQ:
            cp_pos[q + 2] = pltpu.async_copy(
                pos_hbm.at[pl.ds(t0 + (q + 2) * _C, _C)], ps[q % 2],
                sps[q % 2])

    cp = start(0)
    for k in range(1, _NCHUNK):
        cp_next = start(k)
        process(k - 1, cp)
        cp = cp_next
    process(_NCHUNK - 1, cp)
    for k in range(_NCHUNK - _NG, _NCHUNK):
        cp_out[k].wait()


@jax.jit
def _run(idx, embedding, position_embedding):
    mesh = plsc.VectorSubcoreMesh(
        core_axis_name="c", subcore_axis_name="s", num_cores=_NC,
        num_subcores=_NS)
    out = pl.kernel(
        _body,
        out_type=jax.ShapeDtypeStruct((_N, _D), jnp.float32),
        mesh=mesh,
        scratch_types=[
            pltpu.VMEM((_NCHUNK, _C), jnp.int32),
            pltpu.VMEM((_C, _D), jnp.float32),
            pltpu.VMEM((_C, _D), jnp.float32),
            pltpu.VMEM((_C, _D), jnp.float32),
            pltpu.VMEM((_C, _D), jnp.float32),
            pltpu.VMEM((_C, _D), jnp.float32),
            pltpu.VMEM((_C, _D), jnp.float32),
            pltpu.SemaphoreType.DMA,
            pltpu.SemaphoreType.DMA,
            pltpu.SemaphoreType.DMA,
            pltpu.SemaphoreType.DMA,
            pltpu.SemaphoreType.DMA,
            pltpu.SemaphoreType.DMA,
            pltpu.SemaphoreType.DMA,
            pltpu.SemaphoreType.DMA,
            pltpu.SemaphoreType.DMA,
            pltpu.SemaphoreType.DMA,
        ],
        name="emb_lookup_sc",
    )(idx, embedding, position_embedding)
    return out.reshape(_B, _T, _D)


def kernel(x, embedding, position_embedding):
    idx = (x.astype(jnp.int32)
           .reshape(_B, _NW, _NQ, _C)
           .transpose(1, 2, 0, 3)
           .reshape(_NW, _NCHUNK, _C))
    return _run(idx, embedding, position_embedding)
